# baseline (speedup 1.0000x reference)
import functools

import jax
import jax.numpy as jnp
from jax.experimental import pallas as pl
from jax.experimental.pallas import tpu as pltpu


def _vq_body(flat_ref, cb_ref, q_ref, codes_ref, probs_ref, loss_ref,
             cb2_ref, cn_ref):
    f = flat_ref[:]
    cb = cb_ref[:]
    cb_size = cb.shape[0]

    @pl.when(pl.program_id(0) == 0)
    def _prep():
        cb2_ref[:] = cb + cb
        cn_ref[:] = jnp.sum(cb * cb, axis=1, keepdims=True).T

    m2 = jax.lax.dot_general(
        f, cb2_ref[:], (((1,), (1,)), ((), ())),
        preferred_element_type=jnp.float32,
    )
    fn = jnp.sum(f * f, axis=1, keepdims=True)
    cn = cn_ref[:]
    d = (fn - m2) + cn

    dmin = jnp.min(d, axis=1, keepdims=True)
    e = jnp.exp(dmin - d)
    ssum = jnp.sum(e, axis=1, keepdims=True)
    probs_ref[:] = e * (1.0 / ssum)

    iota_f = jax.lax.broadcasted_iota(jnp.int32, d.shape, 1).astype(jnp.float32)
    cand = jnp.where(d == dmin, iota_f, float(cb_size))
    codes_f = jnp.min(cand, axis=1, keepdims=True)
    codes_ref[:] = codes_f.astype(jnp.int32)

    oh = (iota_f == codes_f).astype(jnp.float32)
    q_ref[:] = jax.lax.dot_general(
        oh, cb, (((1,), (0,)), ((), ())), preferred_element_type=jnp.float32
    )

    part = jnp.sum(dmin).reshape(1, 1)
    i = pl.program_id(0)

    @pl.when(i == 0)
    def _init():
        loss_ref[:] = part

    @pl.when(i > 0)
    def _acc():
        loss_ref[:] = loss_ref[:] + part


@functools.partial(jax.jit, static_argnames=("tile",))
def _vq_pallas(flat, codebook, tile=2048):
    n, d = flat.shape
    cb_size = codebook.shape[0]
    grid = (n // tile,)
    q, codes, probs, loss = pl.pallas_call(
        _vq_body,
        grid=grid,
        in_specs=[
            pl.BlockSpec((tile, d), lambda i: (i, 0)),
            pl.BlockSpec((cb_size, d), lambda i: (0, 0)),
        ],
        out_specs=[
            pl.BlockSpec((tile, d), lambda i: (i, 0)),
            pl.BlockSpec((tile, 1), lambda i: (i, 0)),
            pl.BlockSpec((tile, cb_size), lambda i: (i, 0)),
            pl.BlockSpec((1, 1), lambda i: (0, 0)),
        ],
        out_shape=[
            jax.ShapeDtypeStruct((n, d), jnp.float32),
            jax.ShapeDtypeStruct((n, 1), jnp.int32),
            jax.ShapeDtypeStruct((n, cb_size), jnp.float32),
            jax.ShapeDtypeStruct((1, 1), jnp.float32),
        ],
        scratch_shapes=[
            pltpu.VMEM((cb_size, d), jnp.float32),
            pltpu.VMEM((1, cb_size), jnp.float32),
        ],
    )(flat, codebook)
    return q, codes, probs, loss


def kernel(slot_embeddings, codebook):
    batch, k, d = slot_embeddings.shape
    cb_size = codebook.shape[0]
    flat = slot_embeddings.reshape(-1, d)
    q, codes, probs, loss = _vq_pallas(flat, codebook)
    beta = 0.25
    vq_loss = ((1.0 + beta) * loss[0, 0] / (flat.shape[0] * d)).astype(jnp.float32)
    return (
        q.reshape(batch, k, d),
        codes.reshape(batch, k),
        probs.reshape(batch, k, cb_size),
        vq_loss,
    )


# speedup vs baseline: 1.0276x; 1.0276x over previous
import functools

import jax
import jax.numpy as jnp
from jax.experimental import pallas as pl
from jax.experimental.pallas import tpu as pltpu


def _vq_body(flat_ref, cb_ref, q_ref, codes_ref, probs_ref, loss_ref):
    f = flat_ref[:]
    cb = cb_ref[:]
    cb_size = cb.shape[0]

    m2 = jax.lax.dot_general(
        f + f, cb, (((1,), (1,)), ((), ())), preferred_element_type=jnp.float32
    )
    fn = jnp.sum(f * f, axis=1, keepdims=True)
    cn = jnp.sum(cb * cb, axis=1)
    d = (fn - m2) + cn[None, :]

    dmin = jnp.min(d, axis=1, keepdims=True)
    e = jnp.exp(dmin - d)
    ssum = jnp.sum(e, axis=1, keepdims=True)
    probs_ref[:] = e * (1.0 / ssum)

    iota_f = jax.lax.broadcasted_iota(jnp.int32, d.shape, 1).astype(jnp.float32)
    cand = jnp.where(d == dmin, iota_f, float(cb_size))
    codes_f = jnp.min(cand, axis=1, keepdims=True)
    codes_ref[:] = codes_f.astype(jnp.int32)

    oh = (iota_f == codes_f).astype(jnp.float32)
    q_ref[:] = jax.lax.dot_general(
        oh, cb, (((1,), (0,)), ((), ())), preferred_element_type=jnp.float32
    )

    part = jnp.sum(dmin).reshape(1, 1)
    i = pl.program_id(0)

    @pl.when(i == 0)
    def _init():
        loss_ref[:] = part

    @pl.when(i > 0)
    def _acc():
        loss_ref[:] = loss_ref[:] + part


@functools.partial(jax.jit, static_argnames=("tile",))
def _vq_pallas(flat, codebook, tile=2048):
    n, d = flat.shape
    cb_size = codebook.shape[0]
    grid = (n // tile,)
    q, codes, probs, loss = pl.pallas_call(
        _vq_body,
        grid=grid,
        in_specs=[
            pl.BlockSpec((tile, d), lambda i: (i, 0)),
            pl.BlockSpec((cb_size, d), lambda i: (0, 0)),
        ],
        out_specs=[
            pl.BlockSpec((tile, d), lambda i: (i, 0)),
            pl.BlockSpec((tile, 1), lambda i: (i, 0)),
            pl.BlockSpec((tile, cb_size), lambda i: (i, 0)),
            pl.BlockSpec((1, 1), lambda i: (0, 0)),
        ],
        out_shape=[
            jax.ShapeDtypeStruct((n, d), jnp.float32),
            jax.ShapeDtypeStruct((n, 1), jnp.int32),
            jax.ShapeDtypeStruct((n, cb_size), jnp.float32),
            jax.ShapeDtypeStruct((1, 1), jnp.float32),
        ],
    )(flat, codebook)
    return q, codes, probs, loss


def kernel(slot_embeddings, codebook):
    batch, k, d = slot_embeddings.shape
    cb_size = codebook.shape[0]
    flat = slot_embeddings.reshape(-1, d)
    q, codes, probs, loss = _vq_pallas(flat, codebook)
    beta = 0.25
    vq_loss = ((1.0 + beta) * loss[0, 0] / (flat.shape[0] * d)).astype(jnp.float32)
    return (
        q.reshape(batch, k, d),
        codes.reshape(batch, k),
        probs.reshape(batch, k, cb_size),
        vq_loss,
    )


# speedup vs baseline: 1.0276x; 1.0001x over previous
import functools

import jax
import jax.numpy as jnp
from jax.experimental import pallas as pl
from jax.experimental.pallas import tpu as pltpu


def _vq_body(flat_ref, cb_ref, q_ref, codes_ref, probs_ref, loss_ref):
    f = flat_ref[:]
    cb = cb_ref[:]
    cb_size = cb.shape[0]

    m2 = jax.lax.dot_general(
        f + f, cb, (((1,), (1,)), ((), ())), preferred_element_type=jnp.float32
    )
    fn = jnp.sum(f * f, axis=1, keepdims=True)
    cn = jnp.sum(cb * cb, axis=1)
    d = (fn - m2) + cn[None, :]

    dmin = jnp.min(d, axis=1, keepdims=True)
    e = jnp.exp(dmin - d)
    ssum = jnp.sum(e, axis=1, keepdims=True)
    probs_ref[:] = e * (1.0 / ssum)

    iota_f = jax.lax.broadcasted_iota(jnp.int32, d.shape, 1).astype(jnp.float32)
    cand = jnp.where(d == dmin, iota_f, float(cb_size))
    codes_f = jnp.min(cand, axis=1, keepdims=True)
    codes_ref[:] = codes_f.astype(jnp.int32)

    oh = (iota_f == codes_f).astype(jnp.float32)
    q_ref[:] = jax.lax.dot_general(
        oh, cb, (((1,), (0,)), ((), ())), preferred_element_type=jnp.float32
    )

    part = jnp.sum(dmin).reshape(1, 1)
    i = pl.program_id(0)

    @pl.when(i == 0)
    def _init():
        loss_ref[:] = part

    @pl.when(i > 0)
    def _acc():
        loss_ref[:] = loss_ref[:] + part


@functools.partial(jax.jit, static_argnames=("tile",))
def _vq_pallas(flat, codebook, tile=2304):
    n, d = flat.shape
    cb_size = codebook.shape[0]
    grid = (n // tile,)
    q, codes, probs, loss = pl.pallas_call(
        _vq_body,
        grid=grid,
        in_specs=[
            pl.BlockSpec((tile, d), lambda i: (i, 0)),
            pl.BlockSpec((cb_size, d), lambda i: (0, 0)),
        ],
        out_specs=[
            pl.BlockSpec((tile, d), lambda i: (i, 0)),
            pl.BlockSpec((tile, 1), lambda i: (i, 0)),
            pl.BlockSpec((tile, cb_size), lambda i: (i, 0)),
            pl.BlockSpec((1, 1), lambda i: (0, 0)),
        ],
        out_shape=[
            jax.ShapeDtypeStruct((n, d), jnp.float32),
            jax.ShapeDtypeStruct((n, 1), jnp.int32),
            jax.ShapeDtypeStruct((n, cb_size), jnp.float32),
            jax.ShapeDtypeStruct((1, 1), jnp.float32),
        ],
    )(flat, codebook)
    return q, codes, probs, loss


def kernel(slot_embeddings, codebook):
    batch, k, d = slot_embeddings.shape
    cb_size = codebook.shape[0]
    flat = slot_embeddings.reshape(-1, d)
    q, codes, probs, loss = _vq_pallas(flat, codebook)
    beta = 0.25
    vq_loss = ((1.0 + beta) * loss[0, 0] / (flat.shape[0] * d)).astype(jnp.float32)
    return (
        q.reshape(batch, k, d),
        codes.reshape(batch, k),
        probs.reshape(batch, k, cb_size),
        vq_loss,
    )
